# baseline (speedup 1.0000x reference)
import functools
import math

import jax
import jax.numpy as jnp
from jax import lax
from jax.experimental import pallas as pl
from jax.experimental.pallas import tpu as pltpu
from jax.experimental.pallas import tpu_sc as plsc

E = 16
K = 2
H = 1024
F = 2048
CAPF = 1.25

TB = 256
FB = 512
CB = 512

NC, NS = 2, 16
NW = NC * NS


def _router_body(cap, nslot, x_ref, rw_ref, scale_ref, bias_ref,
                 xn_ref, sa_ref, sb_ref, wa_ref, wb_ref, counts_ref):
    i = pl.program_id(0)

    @pl.when(i == 0)
    def _():
        counts_ref[...] = jnp.zeros_like(counts_ref)

    xb = x_ref[...]
    mu = jnp.mean(xb, axis=-1, keepdims=True)
    xc = xb - mu
    var = jnp.mean(xc * xc, axis=-1, keepdims=True)
    xn = xc * jax.lax.rsqrt(var + 1e-5) * scale_ref[...] + bias_ref[...]
    xn_ref[...] = xn

    logits = jnp.dot(xn, rw_ref[...], preferred_element_type=jnp.float32)
    m = jnp.max(logits, axis=-1, keepdims=True)
    p = jnp.exp(logits - m)
    probs = p / jnp.sum(p, axis=-1, keepdims=True)

    iota = jax.lax.broadcasted_iota(jnp.int32, (TB, E), 1)
    v1 = jnp.max(probs, axis=-1, keepdims=True)
    a1 = jnp.min(jnp.where(probs == v1, iota, E), axis=-1, keepdims=True)
    probs2 = jnp.where(iota == a1, -1.0, probs)
    v2 = jnp.max(probs2, axis=-1, keepdims=True)
    a2 = jnp.min(jnp.where(probs2 == v2, iota, E), axis=-1, keepdims=True)
    ws = v1 + v2
    w1 = v1 / ws
    w2 = v2 / ws

    onehot = jnp.logical_or(iota == a1, iota == a2).astype(jnp.float32)
    r = jax.lax.broadcasted_iota(jnp.int32, (TB, TB), 0)
    c = jax.lax.broadcasted_iota(jnp.int32, (TB, TB), 1)
    tri = (r >= c).astype(jnp.float32)
    csum = jnp.dot(tri, onehot, preferred_element_type=jnp.float32,
                   precision=jax.lax.Precision.HIGHEST)
    base = counts_ref[...]
    pos = base + csum - 1.0
    counts_ref[...] = base + csum[TB - 1:TB, :]

    pos1 = jnp.sum(jnp.where(iota == a1, pos, 0.0), axis=-1, keepdims=True)
    pos2 = jnp.sum(jnp.where(iota == a2, pos, 0.0), axis=-1, keepdims=True)
    keep1 = pos1 < cap
    keep2 = pos2 < cap
    sa_ref[...] = jnp.where(keep1, a1 * cap + pos1.astype(jnp.int32), nslot)
    sb_ref[...] = jnp.where(keep2, a2 * cap + pos2.astype(jnp.int32), nslot)
    wa_ref[...] = jnp.where(keep1, w1, 0.0)
    wb_ref[...] = jnp.where(keep2, w2, 0.0)


def _router(x2d, rw, scale, bias, cap, nslot):
    t = x2d.shape[0]
    grid = (t // TB,)
    return pl.pallas_call(
        functools.partial(_router_body, cap, nslot),
        grid=grid,
        in_specs=[
            pl.BlockSpec((TB, H), lambda i: (i, 0)),
            pl.BlockSpec((H, E), lambda i: (0, 0)),
            pl.BlockSpec((1, H), lambda i: (0, 0)),
            pl.BlockSpec((1, H), lambda i: (0, 0)),
        ],
        out_specs=[
            pl.BlockSpec((TB, H), lambda i: (i, 0)),
            pl.BlockSpec((TB, 1), lambda i: (i, 0)),
            pl.BlockSpec((TB, 1), lambda i: (i, 0)),
            pl.BlockSpec((TB, 1), lambda i: (i, 0)),
            pl.BlockSpec((TB, 1), lambda i: (i, 0)),
        ],
        out_shape=[
            jax.ShapeDtypeStruct((t, H), jnp.float32),
            jax.ShapeDtypeStruct((t, 1), jnp.int32),
            jax.ShapeDtypeStruct((t, 1), jnp.int32),
            jax.ShapeDtypeStruct((t, 1), jnp.float32),
            jax.ShapeDtypeStruct((t, 1), jnp.float32),
        ],
        scratch_shapes=[pltpu.VMEM((1, E), jnp.float32)],
        compiler_params=pltpu.CompilerParams(
            dimension_semantics=("arbitrary",)),
    )(x2d, rw, scale, bias)


def _ffn_body(nj, cap, xin_ref, wg_ref, wu_ref, wd_ref, out_ref, acc_ref):
    j = pl.program_id(1)

    @pl.when(j == 0)
    def _():
        acc_ref[...] = jnp.zeros_like(acc_ref)

    xb = xin_ref[0]
    g = jnp.dot(xb, wg_ref[0], preferred_element_type=jnp.float32)
    u = jnp.dot(xb, wu_ref[0], preferred_element_type=jnp.float32)
    hmid = g * jax.nn.sigmoid(g) * u
    acc_ref[...] += jnp.dot(hmid, wd_ref[0], preferred_element_type=jnp.float32)

    @pl.when(j == nj - 1)
    def _():
        out_ref[0] = acc_ref[...]


def _ffn(xin, wg, wu, wd, cap):
    nj = F // FB
    return pl.pallas_call(
        functools.partial(_ffn_body, nj, cap),
        grid=(E, nj),
        in_specs=[
            pl.BlockSpec((1, cap, H), lambda e, j: (e, 0, 0)),
            pl.BlockSpec((1, H, FB), lambda e, j: (e, 0, j)),
            pl.BlockSpec((1, H, FB), lambda e, j: (e, 0, j)),
            pl.BlockSpec((1, FB, H), lambda e, j: (e, j, 0)),
        ],
        out_specs=pl.BlockSpec((1, cap, H), lambda e, j: (e, 0, 0)),
        out_shape=jax.ShapeDtypeStruct((E, cap, H), jnp.float32),
        scratch_shapes=[pltpu.VMEM((cap, H), jnp.float32)],
        compiler_params=pltpu.CompilerParams(
            dimension_semantics=("arbitrary", "arbitrary")),
    )(xin, wg, wu, wd)


def _mesh():
    return plsc.VectorSubcoreMesh(
        core_axis_name="c", subcore_axis_name="s",
        num_cores=NC, num_subcores=NS)


def _dispatch(xn, sa, sb, t, nslot):
    spw = nslot // NW
    nch = 4
    ch = spw // nch

    @functools.partial(
        pl.kernel, mesh=_mesh(),
        out_type=[
            jax.ShapeDtypeStruct((nslot, H), jnp.float32),
            jax.ShapeDtypeStruct((nslot,), jnp.int32),
        ],
        scratch_types=[
            pltpu.VMEM((t,), jnp.int32),
            pltpu.VMEM((t,), jnp.int32),
            pltpu.VMEM((nslot + 16,), jnp.int32),
            pltpu.VMEM((nslot + 16,), jnp.int32),
            pltpu.VMEM((ch, H), jnp.float32),
            pltpu.SemaphoreType.DMA,
        ],
        compiler_params=pltpu.CompilerParams(needs_layout_passes=False),
    )
    def k(xn_hbm, sa_hbm, sb_hbm, out_hbm, sel2_hbm, sa_v, sb_v, sel_v,
          sel2_v, rows_v, sem):
        wid = lax.axis_index("s") * NC + lax.axis_index("c")
        pltpu.sync_copy(sa_hbm, sa_v)
        pltpu.sync_copy(sb_hbm, sb_v)

        def init(i, carry):
            sel_v[pl.ds(i * 16, 16)] = jnp.zeros((16,), jnp.int32)
            sel2_v[pl.ds(i * 16, 16)] = jnp.full((16,), t, jnp.int32)
            return carry
        lax.fori_loop(0, (nslot + 16) // 16, init, 0)

        def scat(i, carry):
            toks = i * 16 + lax.iota(jnp.int32, 16)
            va = sa_v[pl.ds(i * 16, 16)]
            vb = sb_v[pl.ds(i * 16, 16)]
            plsc.store_scatter(sel_v, [va], toks)
            plsc.store_scatter(sel_v, [vb], toks)
            plsc.store_scatter(sel2_v, [va], toks)
            plsc.store_scatter(sel2_v, [vb], toks + (t + 8))
            return carry
        lax.fori_loop(0, t // 16, scat, 0)

        base = wid * spw
        pltpu.sync_copy(sel2_v.at[pl.ds(base, spw)],
                        sel2_hbm.at[pl.ds(base, spw)])
        for c in range(nch):
            pltpu.async_copy(
                xn_hbm.at[sel_v.at[pl.ds(base + c * ch, ch)]],
                rows_v, sem).wait()
            pltpu.sync_copy(rows_v, out_hbm.at[pl.ds(base + c * ch, ch)])

    return k(xn, sa, sb)


def _unpermute(eo, sel2, t, nslot):
    spw = nslot // NW
    nch = 4
    ch = spw // nch

    @functools.partial(
        pl.kernel, mesh=_mesh(),
        out_type=jax.ShapeDtypeStruct((2 * (t + 8), H), jnp.float32),
        scratch_types=[
            [pltpu.VMEM((ch,), jnp.int32) for _ in range(2)],
            [pltpu.VMEM((ch, H), jnp.float32) for _ in range(2)],
            [pltpu.SemaphoreType.DMA for _ in range(2)],
            [pltpu.SemaphoreType.DMA for _ in range(2)],
        ],
        compiler_params=pltpu.CompilerParams(needs_layout_passes=False),
    )
    def k(eo_hbm, sel2_hbm, go_hbm, idx_v, rows_v, semg, semw):
        wid = lax.axis_index("s") * NC + lax.axis_index("c")
        base = wid * spw

        def start(c, s):
            pltpu.sync_copy(sel2_hbm.at[pl.ds(base + c * ch, ch)], idx_v[s])
            return pltpu.async_copy(eo_hbm.at[pl.ds(base + c * ch, ch)],
                                    rows_v[s], semg[s])

        pend = {0: start(0, 0)}
        writes = {}
        for c in range(nch):
            s = c % 2
            pend.pop(c).wait()
            if c + 1 < nch:
                if c - 1 in writes:
                    writes.pop(c - 1).wait()
                pend[c + 1] = start(c + 1, (c + 1) % 2)
            writes[c] = pltpu.async_copy(rows_v[s], go_hbm.at[idx_v[s]],
                                         semw[s])
        for cp in writes.values():
            cp.wait()

    return k(eo, sel2)


def _comb_body(x_ref, go_ref, wa_ref, wb_ref, out_ref):
    ga = go_ref[0]
    gb = go_ref[1]
    wa = wa_ref[...]
    wb = wb_ref[...]
    out_ref[...] = x_ref[...] \
        + jnp.where(wa > 0, wa * ga, 0.0) \
        + jnp.where(wb > 0, wb * gb, 0.0)


def _combine(x2d, go3, wa, wb, t):
    return pl.pallas_call(
        _comb_body,
        grid=(t // CB,),
        in_specs=[
            pl.BlockSpec((CB, H), lambda i: (i, 0)),
            pl.BlockSpec((2, CB, H), lambda i: (0, i, 0)),
            pl.BlockSpec((CB, 1), lambda i: (i, 0)),
            pl.BlockSpec((CB, 1), lambda i: (i, 0)),
        ],
        out_specs=pl.BlockSpec((CB, H), lambda i: (i, 0)),
        out_shape=jax.ShapeDtypeStruct((t, H), jnp.float32),
        compiler_params=pltpu.CompilerParams(
            dimension_semantics=("arbitrary",)),
    )(x2d, go3, wa, wb)


def kernel(x, router_weight, w_gate, w_up, w_down, ln_scale, ln_bias):
    b, s, _ = x.shape
    t = b * s
    cap = int(math.ceil(CAPF * t / E))
    nslot = E * cap

    x2d = x.reshape(t, H)
    xn, sa, sb, wa, wb = _router(
        x2d, router_weight, ln_scale.reshape(1, H), ln_bias.reshape(1, H),
        cap, nslot)
    sa = sa.reshape(t)
    sb = sb.reshape(t)

    xin, sel2 = _dispatch(xn, sa, sb, t, nslot)
    eo = _ffn(xin.reshape(E, cap, H), w_gate, w_up, w_down, cap)
    go = _unpermute(eo.reshape(nslot, H), sel2, t, nslot)
    go3 = go.reshape(2, t + 8, H)
    out = _combine(x2d, go3, wa, wb, t)
    return out.reshape(b, s, H)
